# Initial kernel scaffold; baseline (speedup 1.0000x reference)
import functools

import jax
import jax.numpy as jnp
from jax import lax
from jax.experimental import pallas as pl
from jax.experimental.pallas import tpu as pltpu
from jax.experimental.pallas import tpu_sc as plsc

N = 10000
E = 320000
NC, NS = 2, 16
NW = NC * NS
CHUNK = 128
EPT = 10240
NCHUNK = EPT // CHUNK
ACC = 10240
RPT = ACC // NS
BL = 1000


def _make_mesh():
    return plsc.VectorSubcoreMesh(
        core_axis_name="c", subcore_axis_name="s", num_cores=NC, num_subcores=NS
    )


def _make_agg(F):
    @functools.partial(
        pl.kernel,
        out_type=jax.ShapeDtypeStruct((NC, ACC, F), jnp.float32),
        mesh=_make_mesh(),
        scratch_types=[
            pltpu.VMEM((NCHUNK, CHUNK), jnp.int32),
            pltpu.VMEM((NCHUNK, CHUNK), jnp.int32),
            pltpu.VMEM((CHUNK, F), jnp.float32),
            pltpu.VMEM((RPT, F), jnp.float32),
            pltpu.VMEM_SHARED((ACC, F), jnp.float32),
            pltpu.SemaphoreType.DMA,
        ],
    )
    def agg(hp, srcr, dstr, zeros, out, src_v, dst_v, gbuf, zbuf, acc, gsem):
        c = lax.axis_index("c")
        s = lax.axis_index("s")
        wid = s * NC + c
        pltpu.sync_copy(zeros, zbuf)
        pltpu.sync_copy(zbuf, acc.at[pl.ds(s * RPT, RPT)])
        pltpu.sync_copy(srcr.at[wid], src_v)
        pltpu.sync_copy(dstr.at[wid], dst_v)
        plsc.subcore_barrier()

        def body(j, carry):
            pltpu.async_copy(hp.at[src_v.at[j]], gbuf, gsem).wait()
            pltpu.sync_copy(gbuf, acc.at[dst_v.at[j]], add=True)
            return carry

        lax.fori_loop(0, NCHUNK, body, 0)
        plsc.subcore_barrier()
        pltpu.sync_copy(acc.at[pl.ds(s * RPT, RPT)],
                        out.at[c, pl.ds(s * RPT, RPT)])

    return agg


def _make_deg():
    @functools.partial(
        pl.kernel,
        out_type=jax.ShapeDtypeStruct((NC, ACC, 16), jnp.float32),
        mesh=_make_mesh(),
        scratch_types=[
            pltpu.VMEM((NCHUNK, CHUNK), jnp.int32),
            pltpu.VMEM((CHUNK, 16), jnp.float32),
            pltpu.VMEM((RPT, 16), jnp.float32),
            pltpu.VMEM_SHARED((ACC, 16), jnp.float32),
        ],
    )
    def deg(dstr, ones, zeros, out, dst_v, ones_v, zbuf, acc):
        c = lax.axis_index("c")
        s = lax.axis_index("s")
        wid = s * NC + c
        pltpu.sync_copy(zeros, zbuf)
        pltpu.sync_copy(zbuf, acc.at[pl.ds(s * RPT, RPT)])
        pltpu.sync_copy(dstr.at[wid], dst_v)
        pltpu.sync_copy(ones, ones_v)
        plsc.subcore_barrier()

        def body(j, carry):
            pltpu.sync_copy(ones_v, acc.at[dst_v.at[j]], add=True)
            return carry

        lax.fori_loop(0, NCHUNK, body, 0)
        plsc.subcore_barrier()
        pltpu.sync_copy(acc.at[pl.ds(s * RPT, RPT)],
                        out.at[c, pl.ds(s * RPT, RPT)])

    return deg


_deg_kernel = _make_deg()
_agg32 = _make_agg(32)
_agg16 = _make_agg(16)


def _tc_first(x, W1, degp):
    def body(x_ref, w_ref, dp_ref, h_ref, dv_ref):
        deg = jnp.sum(dp_ref[...], axis=(0, 2)) + 1.0
        dinv = lax.rsqrt(deg)
        h = jnp.dot(x_ref[...], w_ref[...], preferred_element_type=jnp.float32)
        h_ref[...] = h * dinv[:, None]
        dv_ref[...] = jnp.broadcast_to(dinv[:, None], (BL, 16))

    return pl.pallas_call(
        body,
        grid=(N // BL,),
        in_specs=[
            pl.BlockSpec((BL, 128), lambda i: (i, 0)),
            pl.BlockSpec((128, 32), lambda i: (0, 0)),
            pl.BlockSpec((2, BL, 16), lambda i: (0, i, 0)),
        ],
        out_specs=[
            pl.BlockSpec((BL, 32), lambda i: (i, 0)),
            pl.BlockSpec((BL, 16), lambda i: (i, 0)),
        ],
        out_shape=[
            jax.ShapeDtypeStruct((N, 32), jnp.float32),
            jax.ShapeDtypeStruct((N, 16), jnp.float32),
        ],
    )(x, W1, degp)


def _tc_mid(p, hp, dinv16, b, W, Fi, Fo):
    def body(p_ref, h_ref, dv_ref, b_ref, w_ref, o_ref):
        dinv = dv_ref[:, :1]
        t = (p_ref[0] + p_ref[1] + h_ref[...]) * dinv + b_ref[...]
        r = jnp.maximum(t, 0.0)
        o_ref[...] = (
            jnp.dot(r, w_ref[...], preferred_element_type=jnp.float32) * dinv
        )

    return pl.pallas_call(
        body,
        grid=(N // BL,),
        in_specs=[
            pl.BlockSpec((2, BL, Fi), lambda i: (0, i, 0)),
            pl.BlockSpec((BL, Fi), lambda i: (i, 0)),
            pl.BlockSpec((BL, 16), lambda i: (i, 0)),
            pl.BlockSpec((1, Fi), lambda i: (0, 0)),
            pl.BlockSpec((Fi, Fo), lambda i: (0, 0)),
        ],
        out_specs=pl.BlockSpec((BL, Fo), lambda i: (i, 0)),
        out_shape=jax.ShapeDtypeStruct((N, Fo), jnp.float32),
    )(p, hp, dinv16, b, W)


def _tc_last(p, hp, dinv16, b):
    def body(p_ref, h_ref, dv_ref, b_ref, o_ref):
        dinv = dv_ref[:, :1]
        o_ref[...] = (p_ref[0] + p_ref[1] + h_ref[...]) * dinv + b_ref[...]

    return pl.pallas_call(
        body,
        grid=(N // BL,),
        in_specs=[
            pl.BlockSpec((2, BL, 16), lambda i: (0, i, 0)),
            pl.BlockSpec((BL, 16), lambda i: (i, 0)),
            pl.BlockSpec((BL, 16), lambda i: (i, 0)),
            pl.BlockSpec((1, 16), lambda i: (0, 0)),
        ],
        out_specs=pl.BlockSpec((BL, 16), lambda i: (i, 0)),
        out_shape=jax.ShapeDtypeStruct((N, 16), jnp.float32),
    )(p, hp, dinv16, b)


def kernel(x, edge_index, W1, b1, Wn, bn, W2, b2):
    src = edge_index[0]
    dst = edge_index[1]
    pad = NW * EPT - E
    src_p = jnp.concatenate([src, jnp.zeros((pad,), jnp.int32)])
    dst_p = jnp.concatenate([dst, jnp.full((pad,), N, jnp.int32)])
    srcr = src_p.reshape(NW, NCHUNK, CHUNK)
    dstr = dst_p.reshape(NW, NCHUNK, CHUNK)
    ones16 = jnp.ones((CHUNK, 16), jnp.float32)
    zeros16 = jnp.zeros((RPT, 16), jnp.float32)
    zeros32 = jnp.zeros((RPT, 32), jnp.float32)

    degp = _deg_kernel(dstr, ones16, zeros16)[:, :N, :]
    h1p, dinv16 = _tc_first(x, W1, degp)
    p1 = _agg32(h1p, srcr, dstr, zeros32)[:, :N, :]
    h2p = _tc_mid(p1, h1p, dinv16, b1.reshape(1, -1), Wn, 32, 32)
    p2 = _agg32(h2p, srcr, dstr, zeros32)[:, :N, :]
    h3p = _tc_mid(p2, h2p, dinv16, bn.reshape(1, -1), W2, 32, 16)
    p3 = _agg16(h3p, srcr, dstr, zeros16)[:, :N, :]
    return _tc_last(p3, h3p, dinv16, b2.reshape(1, -1))


# speedup vs baseline: 18.1233x; 18.1233x over previous
import functools

import jax
import jax.numpy as jnp
from jax import lax
from jax.experimental import pallas as pl
from jax.experimental.pallas import tpu as pltpu
from jax.experimental.pallas import tpu_sc as plsc

N = 10000
E = 320000
NC, NS = 2, 16
NW = NC * NS
CHUNK = 128
EPT = 10240
NCHUNK = EPT // CHUNK
ACC = 10240
RPT = ACC // NS
BL = 1000


def _make_mesh():
    return plsc.VectorSubcoreMesh(
        core_axis_name="c", subcore_axis_name="s", num_cores=NC, num_subcores=NS
    )


def _make_agg(F):
    @functools.partial(
        pl.kernel,
        out_type=jax.ShapeDtypeStruct((NC, ACC, F), jnp.float32),
        mesh=_make_mesh(),
        scratch_types=[
            pltpu.VMEM((NCHUNK, CHUNK), jnp.int32),
            pltpu.VMEM((NCHUNK, CHUNK), jnp.int32),
            pltpu.VMEM((CHUNK, F), jnp.float32),
            pltpu.VMEM((RPT, F), jnp.float32),
            pltpu.VMEM_SHARED((ACC, F), jnp.float32),
            pltpu.SemaphoreType.DMA,
        ],
        compiler_params=pltpu.CompilerParams(use_tc_tiling_on_sc=False),
    )
    def agg(hp, srcr, dstr, zeros, out, src_v, dst_v, gbuf, zbuf, acc, gsem):
        c = lax.axis_index("c")
        s = lax.axis_index("s")
        wid = s * NC + c
        pltpu.sync_copy(zeros, zbuf)
        pltpu.sync_copy(zbuf, acc.at[pl.ds(s * RPT, RPT)])
        pltpu.sync_copy(srcr.at[wid], src_v)
        pltpu.sync_copy(dstr.at[wid], dst_v)
        plsc.subcore_barrier()

        def body(j, carry):
            pltpu.async_copy(hp.at[src_v.at[j]], gbuf, gsem).wait()
            pltpu.sync_copy(gbuf, acc.at[dst_v.at[j]], add=True)
            return carry

        lax.fori_loop(0, NCHUNK, body, 0)
        plsc.subcore_barrier()
        pltpu.sync_copy(acc.at[pl.ds(s * RPT, RPT)],
                        out.at[c, pl.ds(s * RPT, RPT)])

    return agg


def _make_deg():
    @functools.partial(
        pl.kernel,
        out_type=jax.ShapeDtypeStruct((NC, ACC, 16), jnp.float32),
        mesh=_make_mesh(),
        scratch_types=[
            pltpu.VMEM((NCHUNK, CHUNK), jnp.int32),
            pltpu.VMEM((CHUNK, 16), jnp.float32),
            pltpu.VMEM((RPT, 16), jnp.float32),
            pltpu.VMEM_SHARED((ACC, 16), jnp.float32),
        ],
        compiler_params=pltpu.CompilerParams(use_tc_tiling_on_sc=False),
    )
    def deg(dstr, ones, zeros, out, dst_v, ones_v, zbuf, acc):
        c = lax.axis_index("c")
        s = lax.axis_index("s")
        wid = s * NC + c
        pltpu.sync_copy(zeros, zbuf)
        pltpu.sync_copy(zbuf, acc.at[pl.ds(s * RPT, RPT)])
        pltpu.sync_copy(dstr.at[wid], dst_v)
        pltpu.sync_copy(ones, ones_v)
        plsc.subcore_barrier()

        def body(j, carry):
            pltpu.sync_copy(ones_v, acc.at[dst_v.at[j]], add=True)
            return carry

        lax.fori_loop(0, NCHUNK, body, 0)
        plsc.subcore_barrier()
        pltpu.sync_copy(acc.at[pl.ds(s * RPT, RPT)],
                        out.at[c, pl.ds(s * RPT, RPT)])

    return deg


_deg_kernel = _make_deg()
_agg32 = _make_agg(32)
_agg16 = _make_agg(16)


def _tc_first(x, W1, degp):
    def body(x_ref, w_ref, dp_ref, h_ref, dv_ref):
        deg = jnp.sum(dp_ref[..., 0], axis=0) + 1.0
        dinv = lax.rsqrt(deg)
        h = jnp.dot(x_ref[...], w_ref[...], preferred_element_type=jnp.float32)
        h_ref[...] = h * dinv[:, None]
        dv_ref[...] = jnp.broadcast_to(dinv[:, None], (BL, 16))

    return pl.pallas_call(
        body,
        grid=(N // BL,),
        in_specs=[
            pl.BlockSpec((BL, 128), lambda i: (i, 0)),
            pl.BlockSpec((128, 32), lambda i: (0, 0)),
            pl.BlockSpec((2, BL, 16), lambda i: (0, i, 0)),
        ],
        out_specs=[
            pl.BlockSpec((BL, 32), lambda i: (i, 0)),
            pl.BlockSpec((BL, 16), lambda i: (i, 0)),
        ],
        out_shape=[
            jax.ShapeDtypeStruct((N, 32), jnp.float32),
            jax.ShapeDtypeStruct((N, 16), jnp.float32),
        ],
    )(x, W1, degp)


def _tc_mid(p, hp, dinv16, b, W, Fi, Fo):
    def body(p_ref, h_ref, dv_ref, b_ref, w_ref, o_ref):
        dinv = dv_ref[:, :1]
        t = (p_ref[0] + p_ref[1] + h_ref[...]) * dinv + b_ref[...]
        r = jnp.maximum(t, 0.0)
        o_ref[...] = (
            jnp.dot(r, w_ref[...], preferred_element_type=jnp.float32) * dinv
        )

    return pl.pallas_call(
        body,
        grid=(N // BL,),
        in_specs=[
            pl.BlockSpec((2, BL, Fi), lambda i: (0, i, 0)),
            pl.BlockSpec((BL, Fi), lambda i: (i, 0)),
            pl.BlockSpec((BL, 16), lambda i: (i, 0)),
            pl.BlockSpec((1, Fi), lambda i: (0, 0)),
            pl.BlockSpec((Fi, Fo), lambda i: (0, 0)),
        ],
        out_specs=pl.BlockSpec((BL, Fo), lambda i: (i, 0)),
        out_shape=jax.ShapeDtypeStruct((N, Fo), jnp.float32),
    )(p, hp, dinv16, b, W)


def _tc_last(p, hp, dinv16, b):
    def body(p_ref, h_ref, dv_ref, b_ref, o_ref):
        dinv = dv_ref[:, :1]
        o_ref[...] = (p_ref[0] + p_ref[1] + h_ref[...]) * dinv + b_ref[...]

    return pl.pallas_call(
        body,
        grid=(N // BL,),
        in_specs=[
            pl.BlockSpec((2, BL, 16), lambda i: (0, i, 0)),
            pl.BlockSpec((BL, 16), lambda i: (i, 0)),
            pl.BlockSpec((BL, 16), lambda i: (i, 0)),
            pl.BlockSpec((1, 16), lambda i: (0, 0)),
        ],
        out_specs=pl.BlockSpec((BL, 16), lambda i: (i, 0)),
        out_shape=jax.ShapeDtypeStruct((N, 16), jnp.float32),
    )(p, hp, dinv16, b)


def kernel(x, edge_index, W1, b1, Wn, bn, W2, b2):
    src = edge_index[0]
    dst = edge_index[1]
    pad = NW * EPT - E
    src_p = jnp.concatenate([src, jnp.zeros((pad,), jnp.int32)])
    dst_p = jnp.concatenate([dst, jnp.full((pad,), N, jnp.int32)])
    srcr = src_p.reshape(NW, NCHUNK, CHUNK)
    dstr = dst_p.reshape(NW, NCHUNK, CHUNK)
    ones16 = jnp.ones((CHUNK, 16), jnp.float32)
    zeros16 = jnp.zeros((RPT, 16), jnp.float32)
    zeros32 = jnp.zeros((RPT, 32), jnp.float32)

    degp = _deg_kernel(dstr, ones16, zeros16)[:, :N, :]
    h1p, dinv16 = _tc_first(x, W1, degp)
    p1 = _agg32(h1p, srcr, dstr, zeros32)[:, :N, :]
    h2p = _tc_mid(p1, h1p, dinv16, b1.reshape(1, -1), Wn, 32, 32)
    p2 = _agg32(h2p, srcr, dstr, zeros32)[:, :N, :]
    h3p = _tc_mid(p2, h2p, dinv16, bn.reshape(1, -1), W2, 32, 16)
    p3 = _agg16(h3p, srcr, dstr, zeros16)[:, :N, :]
    return _tc_last(p3, h3p, dinv16, b2.reshape(1, -1))


# speedup vs baseline: 22.5920x; 1.2466x over previous
import functools

import jax
import jax.numpy as jnp
from jax import lax
from jax.experimental import pallas as pl
from jax.experimental.pallas import tpu as pltpu
from jax.experimental.pallas import tpu_sc as plsc

N = 10000
E = 320000
NC, NS = 2, 16
NW = NC * NS
CHUNK = 128
EPT = 10240
NCHUNK = EPT // CHUNK
ACC = 10240
RPT = ACC // NS
BL = 1000


def _make_mesh():
    return plsc.VectorSubcoreMesh(
        core_axis_name="c", subcore_axis_name="s", num_cores=NC, num_subcores=NS
    )


def _make_agg(F):
    @functools.partial(
        pl.kernel,
        out_type=jax.ShapeDtypeStruct((NC, ACC, F), jnp.float32),
        mesh=_make_mesh(),
        scratch_types=[
            pltpu.VMEM((NCHUNK, CHUNK), jnp.int32),
            pltpu.VMEM((NCHUNK, CHUNK), jnp.int32),
            pltpu.VMEM((2, CHUNK, F), jnp.float32),
            pltpu.VMEM((RPT, F), jnp.float32),
            pltpu.VMEM_SHARED((ACC, F), jnp.float32),
            pltpu.SemaphoreType.DMA,
            pltpu.SemaphoreType.DMA,
        ],
        compiler_params=pltpu.CompilerParams(use_tc_tiling_on_sc=False),
    )
    def agg(hp, srcr, dstr, zeros, out, src_v, dst_v, gbuf, zbuf, acc,
            gsem0, gsem1):
        c = lax.axis_index("c")
        s = lax.axis_index("s")
        wid = s * NC + c
        pltpu.sync_copy(zeros, zbuf)
        pltpu.sync_copy(zbuf, acc.at[pl.ds(s * RPT, RPT)])
        pltpu.sync_copy(srcr.at[wid], src_v)
        pltpu.sync_copy(dstr.at[wid], dst_v)
        plsc.subcore_barrier()

        def start(j, slot, sem):
            pltpu.async_copy(hp.at[src_v.at[j]], gbuf.at[slot], sem)

        def finish(j, slot, sem):
            pltpu.make_async_copy(hp.at[src_v.at[j]], gbuf.at[slot], sem).wait()
            pltpu.sync_copy(gbuf.at[slot], acc.at[dst_v.at[j]], add=True)

        start(0, 0, gsem0)

        def body(k, carry):
            j = 2 * k
            start(j + 1, 1, gsem1)
            finish(j, 0, gsem0)

            @pl.when(j + 2 < NCHUNK)
            def _():
                start(j + 2, 0, gsem0)

            finish(j + 1, 1, gsem1)
            return carry

        lax.fori_loop(0, NCHUNK // 2, body, 0)
        plsc.subcore_barrier()
        pltpu.sync_copy(acc.at[pl.ds(s * RPT, RPT)],
                        out.at[c, pl.ds(s * RPT, RPT)])

    return agg


def _make_deg():
    @functools.partial(
        pl.kernel,
        out_type=jax.ShapeDtypeStruct((NC, ACC, 16), jnp.float32),
        mesh=_make_mesh(),
        scratch_types=[
            pltpu.VMEM((NCHUNK, CHUNK), jnp.int32),
            pltpu.VMEM((CHUNK, 16), jnp.float32),
            pltpu.VMEM((RPT, 16), jnp.float32),
            pltpu.VMEM_SHARED((ACC, 16), jnp.float32),
        ],
        compiler_params=pltpu.CompilerParams(use_tc_tiling_on_sc=False),
    )
    def deg(dstr, ones, zeros, out, dst_v, ones_v, zbuf, acc):
        c = lax.axis_index("c")
        s = lax.axis_index("s")
        wid = s * NC + c
        pltpu.sync_copy(zeros, zbuf)
        pltpu.sync_copy(zbuf, acc.at[pl.ds(s * RPT, RPT)])
        pltpu.sync_copy(dstr.at[wid], dst_v)
        pltpu.sync_copy(ones, ones_v)
        plsc.subcore_barrier()

        def body(j, carry):
            pltpu.sync_copy(ones_v, acc.at[dst_v.at[j]], add=True)
            return carry

        lax.fori_loop(0, NCHUNK, body, 0)
        plsc.subcore_barrier()
        pltpu.sync_copy(acc.at[pl.ds(s * RPT, RPT)],
                        out.at[c, pl.ds(s * RPT, RPT)])

    return deg


_deg_kernel = _make_deg()
_agg32 = _make_agg(32)
_agg16 = _make_agg(16)


def _tc_first(x, W1, degp):
    def body(x_ref, w_ref, dp_ref, h_ref, dv_ref):
        deg = jnp.sum(dp_ref[..., 0], axis=0) + 1.0
        dinv = lax.rsqrt(deg)
        h = jnp.dot(x_ref[...], w_ref[...], preferred_element_type=jnp.float32)
        h_ref[...] = h * dinv[:, None]
        dv_ref[...] = jnp.broadcast_to(dinv[:, None], (BL, 16))

    return pl.pallas_call(
        body,
        grid=(N // BL,),
        in_specs=[
            pl.BlockSpec((BL, 128), lambda i: (i, 0)),
            pl.BlockSpec((128, 32), lambda i: (0, 0)),
            pl.BlockSpec((2, BL, 16), lambda i: (0, i, 0)),
        ],
        out_specs=[
            pl.BlockSpec((BL, 32), lambda i: (i, 0)),
            pl.BlockSpec((BL, 16), lambda i: (i, 0)),
        ],
        out_shape=[
            jax.ShapeDtypeStruct((N, 32), jnp.float32),
            jax.ShapeDtypeStruct((N, 16), jnp.float32),
        ],
    )(x, W1, degp)


def _tc_mid(p, hp, dinv16, b, W, Fi, Fo):
    def body(p_ref, h_ref, dv_ref, b_ref, w_ref, o_ref):
        dinv = dv_ref[:, :1]
        t = (p_ref[0] + p_ref[1] + h_ref[...]) * dinv + b_ref[...]
        r = jnp.maximum(t, 0.0)
        o_ref[...] = (
            jnp.dot(r, w_ref[...], preferred_element_type=jnp.float32) * dinv
        )

    return pl.pallas_call(
        body,
        grid=(N // BL,),
        in_specs=[
            pl.BlockSpec((2, BL, Fi), lambda i: (0, i, 0)),
            pl.BlockSpec((BL, Fi), lambda i: (i, 0)),
            pl.BlockSpec((BL, 16), lambda i: (i, 0)),
            pl.BlockSpec((1, Fi), lambda i: (0, 0)),
            pl.BlockSpec((Fi, Fo), lambda i: (0, 0)),
        ],
        out_specs=pl.BlockSpec((BL, Fo), lambda i: (i, 0)),
        out_shape=jax.ShapeDtypeStruct((N, Fo), jnp.float32),
    )(p, hp, dinv16, b, W)


def _tc_last(p, hp, dinv16, b):
    def body(p_ref, h_ref, dv_ref, b_ref, o_ref):
        dinv = dv_ref[:, :1]
        o_ref[...] = (p_ref[0] + p_ref[1] + h_ref[...]) * dinv + b_ref[...]

    return pl.pallas_call(
        body,
        grid=(N // BL,),
        in_specs=[
            pl.BlockSpec((2, BL, 16), lambda i: (0, i, 0)),
            pl.BlockSpec((BL, 16), lambda i: (i, 0)),
            pl.BlockSpec((BL, 16), lambda i: (i, 0)),
            pl.BlockSpec((1, 16), lambda i: (0, 0)),
        ],
        out_specs=pl.BlockSpec((BL, 16), lambda i: (i, 0)),
        out_shape=jax.ShapeDtypeStruct((N, 16), jnp.float32),
    )(p, hp, dinv16, b)


def kernel(x, edge_index, W1, b1, Wn, bn, W2, b2):
    src = edge_index[0]
    dst = edge_index[1]
    pad = NW * EPT - E
    src_p = jnp.concatenate([src, jnp.zeros((pad,), jnp.int32)])
    dst_p = jnp.concatenate([dst, jnp.full((pad,), N, jnp.int32)])
    srcr = src_p.reshape(NW, NCHUNK, CHUNK)
    dstr = dst_p.reshape(NW, NCHUNK, CHUNK)
    ones16 = jnp.ones((CHUNK, 16), jnp.float32)
    zeros16 = jnp.zeros((RPT, 16), jnp.float32)
    zeros32 = jnp.zeros((RPT, 32), jnp.float32)

    degp = _deg_kernel(dstr, ones16, zeros16)[:, :N, :]
    h1p, dinv16 = _tc_first(x, W1, degp)
    p1 = _agg32(h1p, srcr, dstr, zeros32)[:, :N, :]
    h2p = _tc_mid(p1, h1p, dinv16, b1.reshape(1, -1), Wn, 32, 32)
    p2 = _agg32(h2p, srcr, dstr, zeros32)[:, :N, :]
    h3p = _tc_mid(p2, h2p, dinv16, bn.reshape(1, -1), W2, 32, 16)
    p3 = _agg16(h3p, srcr, dstr, zeros16)[:, :N, :]
    return _tc_last(p3, h3p, dinv16, b2.reshape(1, -1))


# speedup vs baseline: 38.4931x; 1.7038x over previous
import functools

import jax
import jax.numpy as jnp
from jax import lax
from jax.experimental import pallas as pl
from jax.experimental.pallas import tpu as pltpu
from jax.experimental.pallas import tpu_sc as plsc

N = 10000
E = 320000
NC, NS = 2, 16
NW = NC * NS
CHUNK = 128
EPT = 10240
NCHUNK = EPT // CHUNK
ACC = 10240
RPT = ACC // NS
BL = 1000


def _make_mesh():
    return plsc.VectorSubcoreMesh(
        core_axis_name="c", subcore_axis_name="s", num_cores=NC, num_subcores=NS
    )


def _make_agg(F):
    @functools.partial(
        pl.kernel,
        out_type=jax.ShapeDtypeStruct((NC, ACC, F), jnp.float32),
        mesh=_make_mesh(),
        scratch_types=[
            pltpu.VMEM((NCHUNK, CHUNK), jnp.int32),
            pltpu.VMEM((NCHUNK, CHUNK), jnp.int32),
            pltpu.VMEM((2, CHUNK, F), jnp.float32),
            pltpu.VMEM((RPT, F), jnp.float32),
            pltpu.VMEM_SHARED((ACC, F), jnp.float32),
            pltpu.VMEM_SHARED((N, F), jnp.float32),
            pltpu.SemaphoreType.DMA,
            pltpu.SemaphoreType.DMA,
        ],
        compiler_params=pltpu.CompilerParams(use_tc_tiling_on_sc=False),
    )
    def agg(hp, srcr, dstr, zeros, out, src_v, dst_v, gbuf, zbuf, acc,
            hp_sh, gsem0, gsem1):
        c = lax.axis_index("c")
        s = lax.axis_index("s")
        wid = s * NC + c
        pltpu.sync_copy(zeros, zbuf)
        pltpu.sync_copy(zbuf, acc.at[pl.ds(s * RPT, RPT)])
        pltpu.sync_copy(hp.at[pl.ds(s * (N // NS), N // NS)],
                        hp_sh.at[pl.ds(s * (N // NS), N // NS)])
        pltpu.sync_copy(srcr.at[wid], src_v)
        pltpu.sync_copy(dstr.at[wid], dst_v)
        plsc.subcore_barrier()

        def start(j, slot, sem):
            pltpu.async_copy(hp_sh.at[src_v.at[j]], gbuf.at[slot], sem)

        def finish(j, slot, sem):
            pltpu.make_async_copy(hp_sh.at[src_v.at[j]], gbuf.at[slot],
                                  sem).wait()
            pltpu.sync_copy(gbuf.at[slot], acc.at[dst_v.at[j]], add=True)

        start(0, 0, gsem0)

        def body(k, carry):
            j = 2 * k
            start(j + 1, 1, gsem1)
            finish(j, 0, gsem0)

            @pl.when(j + 2 < NCHUNK)
            def _():
                start(j + 2, 0, gsem0)

            finish(j + 1, 1, gsem1)
            return carry

        lax.fori_loop(0, NCHUNK // 2, body, 0)
        plsc.subcore_barrier()
        pltpu.sync_copy(acc.at[pl.ds(s * RPT, RPT)],
                        out.at[c, pl.ds(s * RPT, RPT)])

    return agg


def _make_deg():
    @functools.partial(
        pl.kernel,
        out_type=jax.ShapeDtypeStruct((NC, ACC, 16), jnp.float32),
        mesh=_make_mesh(),
        scratch_types=[
            pltpu.VMEM((NCHUNK, CHUNK), jnp.int32),
            pltpu.VMEM((CHUNK, 16), jnp.float32),
            pltpu.VMEM((RPT, 16), jnp.float32),
            pltpu.VMEM_SHARED((ACC, 16), jnp.float32),
        ],
        compiler_params=pltpu.CompilerParams(use_tc_tiling_on_sc=False),
    )
    def deg(dstr, ones, zeros, out, dst_v, ones_v, zbuf, acc):
        c = lax.axis_index("c")
        s = lax.axis_index("s")
        wid = s * NC + c
        pltpu.sync_copy(zeros, zbuf)
        pltpu.sync_copy(zbuf, acc.at[pl.ds(s * RPT, RPT)])
        pltpu.sync_copy(dstr.at[wid], dst_v)
        pltpu.sync_copy(ones, ones_v)
        plsc.subcore_barrier()

        def body(j, carry):
            pltpu.sync_copy(ones_v, acc.at[dst_v.at[j]], add=True)
            return carry

        lax.fori_loop(0, NCHUNK, body, 0)
        plsc.subcore_barrier()
        pltpu.sync_copy(acc.at[pl.ds(s * RPT, RPT)],
                        out.at[c, pl.ds(s * RPT, RPT)])

    return deg


_deg_kernel = _make_deg()
_agg32 = _make_agg(32)
_agg16 = _make_agg(16)


def _tc_first(x, W1, degp):
    def body(x_ref, w_ref, dp_ref, h_ref, dv_ref):
        deg = jnp.sum(dp_ref[..., 0], axis=0) + 1.0
        dinv = lax.rsqrt(deg)
        h = jnp.dot(x_ref[...], w_ref[...], preferred_element_type=jnp.float32)
        h_ref[...] = h * dinv[:, None]
        dv_ref[...] = jnp.broadcast_to(dinv[:, None], (BL, 16))

    return pl.pallas_call(
        body,
        grid=(N // BL,),
        in_specs=[
            pl.BlockSpec((BL, 128), lambda i: (i, 0)),
            pl.BlockSpec((128, 32), lambda i: (0, 0)),
            pl.BlockSpec((2, BL, 16), lambda i: (0, i, 0)),
        ],
        out_specs=[
            pl.BlockSpec((BL, 32), lambda i: (i, 0)),
            pl.BlockSpec((BL, 16), lambda i: (i, 0)),
        ],
        out_shape=[
            jax.ShapeDtypeStruct((N, 32), jnp.float32),
            jax.ShapeDtypeStruct((N, 16), jnp.float32),
        ],
    )(x, W1, degp)


def _tc_mid(p, hp, dinv16, b, W, Fi, Fo):
    def body(p_ref, h_ref, dv_ref, b_ref, w_ref, o_ref):
        dinv = dv_ref[:, :1]
        t = (p_ref[0] + p_ref[1] + h_ref[...]) * dinv + b_ref[...]
        r = jnp.maximum(t, 0.0)
        o_ref[...] = (
            jnp.dot(r, w_ref[...], preferred_element_type=jnp.float32) * dinv
        )

    return pl.pallas_call(
        body,
        grid=(N // BL,),
        in_specs=[
            pl.BlockSpec((2, BL, Fi), lambda i: (0, i, 0)),
            pl.BlockSpec((BL, Fi), lambda i: (i, 0)),
            pl.BlockSpec((BL, 16), lambda i: (i, 0)),
            pl.BlockSpec((1, Fi), lambda i: (0, 0)),
            pl.BlockSpec((Fi, Fo), lambda i: (0, 0)),
        ],
        out_specs=pl.BlockSpec((BL, Fo), lambda i: (i, 0)),
        out_shape=jax.ShapeDtypeStruct((N, Fo), jnp.float32),
    )(p, hp, dinv16, b, W)


def _tc_last(p, hp, dinv16, b):
    def body(p_ref, h_ref, dv_ref, b_ref, o_ref):
        dinv = dv_ref[:, :1]
        o_ref[...] = (p_ref[0] + p_ref[1] + h_ref[...]) * dinv + b_ref[...]

    return pl.pallas_call(
        body,
        grid=(N // BL,),
        in_specs=[
            pl.BlockSpec((2, BL, 16), lambda i: (0, i, 0)),
            pl.BlockSpec((BL, 16), lambda i: (i, 0)),
            pl.BlockSpec((BL, 16), lambda i: (i, 0)),
            pl.BlockSpec((1, 16), lambda i: (0, 0)),
        ],
        out_specs=pl.BlockSpec((BL, 16), lambda i: (i, 0)),
        out_shape=jax.ShapeDtypeStruct((N, 16), jnp.float32),
    )(p, hp, dinv16, b)


def kernel(x, edge_index, W1, b1, Wn, bn, W2, b2):
    src = edge_index[0]
    dst = edge_index[1]
    pad = NW * EPT - E
    src_p = jnp.concatenate([src, jnp.zeros((pad,), jnp.int32)])
    dst_p = jnp.concatenate([dst, jnp.full((pad,), N, jnp.int32)])
    srcr = src_p.reshape(NW, NCHUNK, CHUNK)
    dstr = dst_p.reshape(NW, NCHUNK, CHUNK)
    ones16 = jnp.ones((CHUNK, 16), jnp.float32)
    zeros16 = jnp.zeros((RPT, 16), jnp.float32)
    zeros32 = jnp.zeros((RPT, 32), jnp.float32)

    degp = _deg_kernel(dstr, ones16, zeros16)[:, :N, :]
    h1p, dinv16 = _tc_first(x, W1, degp)
    p1 = _agg32(h1p, srcr, dstr, zeros32)[:, :N, :]
    h2p = _tc_mid(p1, h1p, dinv16, b1.reshape(1, -1), Wn, 32, 32)
    p2 = _agg32(h2p, srcr, dstr, zeros32)[:, :N, :]
    h3p = _tc_mid(p2, h2p, dinv16, bn.reshape(1, -1), W2, 32, 16)
    p3 = _agg16(h3p, srcr, dstr, zeros16)[:, :N, :]
    return _tc_last(p3, h3p, dinv16, b2.reshape(1, -1))


# speedup vs baseline: 42.8203x; 1.1124x over previous
import functools

import jax
import jax.numpy as jnp
from jax import lax
from jax.experimental import pallas as pl
from jax.experimental.pallas import tpu as pltpu
from jax.experimental.pallas import tpu_sc as plsc

N = 10000
E = 320000
NC, NS = 2, 16
NW = NC * NS
CHUNK = 128
EPT = 10240
NCHUNK = EPT // CHUNK
ACC = 10240
RPT = ACC // NS
BL = 1000


def _make_mesh():
    return plsc.VectorSubcoreMesh(
        core_axis_name="c", subcore_axis_name="s", num_cores=NC, num_subcores=NS
    )


def _make_agg(F):
    @functools.partial(
        pl.kernel,
        out_type=jax.ShapeDtypeStruct((NC, ACC, F), jnp.float32),
        mesh=_make_mesh(),
        scratch_types=[
            pltpu.VMEM((NCHUNK, CHUNK), jnp.int32),
            pltpu.VMEM((NCHUNK, CHUNK), jnp.int32),
            pltpu.VMEM((4, CHUNK, F), jnp.float32),
            pltpu.VMEM((RPT, F), jnp.float32),
            pltpu.VMEM_SHARED((ACC, F), jnp.float32),
            pltpu.VMEM_SHARED((N, F), jnp.float32),
            [pltpu.SemaphoreType.DMA] * 4,
        ],
        compiler_params=pltpu.CompilerParams(use_tc_tiling_on_sc=False),
    )
    def agg(hp, srcr, dstr, zeros, out, src_v, dst_v, gbuf, zbuf, acc,
            hp_sh, gsems):
        c = lax.axis_index("c")
        s = lax.axis_index("s")
        wid = s * NC + c
        pltpu.sync_copy(zeros, zbuf)
        pltpu.sync_copy(zbuf, acc.at[pl.ds(s * RPT, RPT)])
        pltpu.sync_copy(hp.at[pl.ds(s * (N // NS), N // NS)],
                        hp_sh.at[pl.ds(s * (N // NS), N // NS)])
        pltpu.sync_copy(srcr.at[wid], src_v)
        pltpu.sync_copy(dstr.at[wid], dst_v)
        plsc.subcore_barrier()

        def start(j, slot):
            pltpu.async_copy(hp_sh.at[src_v.at[j]], gbuf.at[slot],
                             gsems[slot])

        def finish(j, slot):
            pltpu.make_async_copy(hp_sh.at[src_v.at[j]], gbuf.at[slot],
                                  gsems[slot]).wait()
            pltpu.sync_copy(gbuf.at[slot], acc.at[dst_v.at[j]], add=True)

        for t in range(4):
            start(t, t)

        def body(k, carry):
            j = 4 * k
            for t in range(4):
                finish(j + t, t)

                @pl.when(j + t + 4 < NCHUNK)
                def _():
                    start(j + t + 4, t)

            return carry

        lax.fori_loop(0, NCHUNK // 4, body, 0)
        plsc.subcore_barrier()
        pltpu.sync_copy(acc.at[pl.ds(s * RPT, RPT)],
                        out.at[c, pl.ds(s * RPT, RPT)])

    return agg


def _make_deg():
    @functools.partial(
        pl.kernel,
        out_type=jax.ShapeDtypeStruct((NC, ACC, 16), jnp.float32),
        mesh=_make_mesh(),
        scratch_types=[
            pltpu.VMEM((NCHUNK, CHUNK), jnp.int32),
            pltpu.VMEM((CHUNK, 16), jnp.float32),
            pltpu.VMEM((RPT, 16), jnp.float32),
            pltpu.VMEM_SHARED((ACC, 16), jnp.float32),
        ],
        compiler_params=pltpu.CompilerParams(use_tc_tiling_on_sc=False),
    )
    def deg(dstr, ones, zeros, out, dst_v, ones_v, zbuf, acc):
        c = lax.axis_index("c")
        s = lax.axis_index("s")
        wid = s * NC + c
        pltpu.sync_copy(zeros, zbuf)
        pltpu.sync_copy(zbuf, acc.at[pl.ds(s * RPT, RPT)])
        pltpu.sync_copy(dstr.at[wid], dst_v)
        pltpu.sync_copy(ones, ones_v)
        plsc.subcore_barrier()

        def body(j, carry):
            pltpu.sync_copy(ones_v, acc.at[dst_v.at[j]], add=True)
            return carry

        lax.fori_loop(0, NCHUNK, body, 0)
        plsc.subcore_barrier()
        pltpu.sync_copy(acc.at[pl.ds(s * RPT, RPT)],
                        out.at[c, pl.ds(s * RPT, RPT)])

    return deg


_deg_kernel = _make_deg()
_agg32 = _make_agg(32)
_agg16 = _make_agg(16)


def _tc_first(x, W1, degp):
    def body(x_ref, w_ref, dp_ref, h_ref, dv_ref):
        deg = jnp.sum(dp_ref[..., 0], axis=0) + 1.0
        dinv = lax.rsqrt(deg)
        h = jnp.dot(x_ref[...], w_ref[...], preferred_element_type=jnp.float32)
        h_ref[...] = h * dinv[:, None]
        dv_ref[...] = jnp.broadcast_to(dinv[:, None], (BL, 16))

    return pl.pallas_call(
        body,
        grid=(N // BL,),
        in_specs=[
            pl.BlockSpec((BL, 128), lambda i: (i, 0)),
            pl.BlockSpec((128, 32), lambda i: (0, 0)),
            pl.BlockSpec((2, BL, 16), lambda i: (0, i, 0)),
        ],
        out_specs=[
            pl.BlockSpec((BL, 32), lambda i: (i, 0)),
            pl.BlockSpec((BL, 16), lambda i: (i, 0)),
        ],
        out_shape=[
            jax.ShapeDtypeStruct((N, 32), jnp.float32),
            jax.ShapeDtypeStruct((N, 16), jnp.float32),
        ],
    )(x, W1, degp)


def _tc_mid(p, hp, dinv16, b, W, Fi, Fo):
    def body(p_ref, h_ref, dv_ref, b_ref, w_ref, o_ref):
        dinv = dv_ref[:, :1]
        t = (p_ref[0] + p_ref[1] + h_ref[...]) * dinv + b_ref[...]
        r = jnp.maximum(t, 0.0)
        o_ref[...] = (
            jnp.dot(r, w_ref[...], preferred_element_type=jnp.float32) * dinv
        )

    return pl.pallas_call(
        body,
        grid=(N // BL,),
        in_specs=[
            pl.BlockSpec((2, BL, Fi), lambda i: (0, i, 0)),
            pl.BlockSpec((BL, Fi), lambda i: (i, 0)),
            pl.BlockSpec((BL, 16), lambda i: (i, 0)),
            pl.BlockSpec((1, Fi), lambda i: (0, 0)),
            pl.BlockSpec((Fi, Fo), lambda i: (0, 0)),
        ],
        out_specs=pl.BlockSpec((BL, Fo), lambda i: (i, 0)),
        out_shape=jax.ShapeDtypeStruct((N, Fo), jnp.float32),
    )(p, hp, dinv16, b, W)


def _tc_last(p, hp, dinv16, b):
    def body(p_ref, h_ref, dv_ref, b_ref, o_ref):
        dinv = dv_ref[:, :1]
        o_ref[...] = (p_ref[0] + p_ref[1] + h_ref[...]) * dinv + b_ref[...]

    return pl.pallas_call(
        body,
        grid=(N // BL,),
        in_specs=[
            pl.BlockSpec((2, BL, 16), lambda i: (0, i, 0)),
            pl.BlockSpec((BL, 16), lambda i: (i, 0)),
            pl.BlockSpec((BL, 16), lambda i: (i, 0)),
            pl.BlockSpec((1, 16), lambda i: (0, 0)),
        ],
        out_specs=pl.BlockSpec((BL, 16), lambda i: (i, 0)),
        out_shape=jax.ShapeDtypeStruct((N, 16), jnp.float32),
    )(p, hp, dinv16, b)


def kernel(x, edge_index, W1, b1, Wn, bn, W2, b2):
    src = edge_index[0]
    dst = edge_index[1]
    pad = NW * EPT - E
    src_p = jnp.concatenate([src, jnp.zeros((pad,), jnp.int32)])
    dst_p = jnp.concatenate([dst, jnp.full((pad,), N, jnp.int32)])
    srcr = src_p.reshape(NW, NCHUNK, CHUNK)
    dstr = dst_p.reshape(NW, NCHUNK, CHUNK)
    ones16 = jnp.ones((CHUNK, 16), jnp.float32)
    zeros16 = jnp.zeros((RPT, 16), jnp.float32)
    zeros32 = jnp.zeros((RPT, 32), jnp.float32)

    degp = _deg_kernel(dstr, ones16, zeros16)
    h1p, dinv16 = _tc_first(x, W1, degp)
    p1 = _agg32(h1p, srcr, dstr, zeros32)
    h2p = _tc_mid(p1, h1p, dinv16, b1.reshape(1, -1), Wn, 32, 32)
    p2 = _agg32(h2p, srcr, dstr, zeros32)
    h3p = _tc_mid(p2, h2p, dinv16, bn.reshape(1, -1), W2, 32, 16)
    p3 = _agg16(h3p, srcr, dstr, zeros16)
    return _tc_last(p3, h3p, dinv16, b2.reshape(1, -1))


# speedup vs baseline: 43.2147x; 1.0092x over previous
import functools

import jax
import jax.numpy as jnp
from jax import lax
from jax.experimental import pallas as pl
from jax.experimental.pallas import tpu as pltpu
from jax.experimental.pallas import tpu_sc as plsc

N = 10000
E = 320000
NC, NS = 2, 16
NW = NC * NS
CHUNK = 128
EPT = 10240
NCHUNK = EPT // CHUNK
ACC = 10240
RPT = ACC // NS
BL = 1000
DF = 8


def _make_mesh():
    return plsc.VectorSubcoreMesh(
        core_axis_name="c", subcore_axis_name="s", num_cores=NC, num_subcores=NS
    )


def _make_agg(F):
    @functools.partial(
        pl.kernel,
        out_type=jax.ShapeDtypeStruct((NC, ACC, F), jnp.float32),
        mesh=_make_mesh(),
        scratch_types=[
            pltpu.VMEM((NCHUNK, CHUNK), jnp.int32),
            pltpu.VMEM((NCHUNK, CHUNK), jnp.int32),
            pltpu.VMEM((4, CHUNK, F), jnp.float32),
            pltpu.VMEM((RPT, F), jnp.float32),
            pltpu.VMEM_SHARED((ACC, F), jnp.float32),
            pltpu.VMEM_SHARED((N, F), jnp.float32),
            [pltpu.SemaphoreType.DMA] * 4,
        ],
        compiler_params=pltpu.CompilerParams(use_tc_tiling_on_sc=False, skip_device_barrier=True),
    )
    def agg(hp, srcr, dstr, zeros, out, src_v, dst_v, gbuf, zbuf, acc,
            hp_sh, gsems):
        c = lax.axis_index("c")
        s = lax.axis_index("s")
        wid = s * NC + c
        pltpu.sync_copy(zeros, zbuf)
        pltpu.sync_copy(zbuf, acc.at[pl.ds(s * RPT, RPT)])
        pltpu.sync_copy(hp.at[pl.ds(s * (N // NS), N // NS)],
                        hp_sh.at[pl.ds(s * (N // NS), N // NS)])
        pltpu.sync_copy(srcr.at[wid], src_v)
        pltpu.sync_copy(dstr.at[wid], dst_v)
        plsc.subcore_barrier()

        def start(j, slot):
            pltpu.async_copy(hp_sh.at[src_v.at[j]], gbuf.at[slot],
                             gsems[slot])

        def finish(j, slot):
            pltpu.make_async_copy(hp_sh.at[src_v.at[j]], gbuf.at[slot],
                                  gsems[slot]).wait()
            pltpu.sync_copy(gbuf.at[slot], acc.at[dst_v.at[j]], add=True)

        for t in range(4):
            start(t, t)

        def body(k, carry):
            j = 4 * k
            for t in range(4):
                finish(j + t, t)

                @pl.when(j + t + 4 < NCHUNK)
                def _():
                    start(j + t + 4, t)

            return carry

        lax.fori_loop(0, NCHUNK // 4, body, 0)
        plsc.subcore_barrier()
        pltpu.sync_copy(acc.at[pl.ds(s * RPT, RPT)],
                        out.at[c, pl.ds(s * RPT, RPT)])

    return agg


def _make_deg():
    @functools.partial(
        pl.kernel,
        out_type=jax.ShapeDtypeStruct((NC, ACC, DF), jnp.float32),
        mesh=_make_mesh(),
        scratch_types=[
            pltpu.VMEM((NCHUNK, CHUNK), jnp.int32),
            pltpu.VMEM((CHUNK, DF), jnp.float32),
            pltpu.VMEM((RPT, DF), jnp.float32),
            pltpu.VMEM_SHARED((ACC, DF), jnp.float32),
        ],
        compiler_params=pltpu.CompilerParams(use_tc_tiling_on_sc=False, skip_device_barrier=True),
    )
    def deg(dstr, ones, zeros, out, dst_v, ones_v, zbuf, acc):
        c = lax.axis_index("c")
        s = lax.axis_index("s")
        wid = s * NC + c
        pltpu.sync_copy(zeros, zbuf)
        pltpu.sync_copy(zbuf, acc.at[pl.ds(s * RPT, RPT)])
        pltpu.sync_copy(dstr.at[wid], dst_v)
        pltpu.sync_copy(ones, ones_v)
        plsc.subcore_barrier()

        def body(j, carry):
            pltpu.sync_copy(ones_v, acc.at[dst_v.at[j]], add=True)
            return carry

        lax.fori_loop(0, NCHUNK, body, 0)
        plsc.subcore_barrier()
        pltpu.sync_copy(acc.at[pl.ds(s * RPT, RPT)],
                        out.at[c, pl.ds(s * RPT, RPT)])

    return deg


_deg_kernel = _make_deg()
_agg32 = _make_agg(32)
_agg16 = _make_agg(16)


def _tc_first(x, W1, degp):
    def body(x_ref, w_ref, dp_ref, h_ref, dv_ref):
        deg = jnp.sum(dp_ref[..., 0], axis=0) + 1.0
        dinv = lax.rsqrt(deg)
        h = jnp.dot(x_ref[...], w_ref[...], preferred_element_type=jnp.float32)
        h_ref[...] = h * dinv[:, None]
        dv_ref[...] = jnp.broadcast_to(dinv[:, None], (BL, 16))

    return pl.pallas_call(
        body,
        grid=(N // BL,),
        in_specs=[
            pl.BlockSpec((BL, 128), lambda i: (i, 0)),
            pl.BlockSpec((128, 32), lambda i: (0, 0)),
            pl.BlockSpec((2, BL, DF), lambda i: (0, i, 0)),
        ],
        out_specs=[
            pl.BlockSpec((BL, 32), lambda i: (i, 0)),
            pl.BlockSpec((BL, 16), lambda i: (i, 0)),
        ],
        out_shape=[
            jax.ShapeDtypeStruct((N, 32), jnp.float32),
            jax.ShapeDtypeStruct((N, 16), jnp.float32),
        ],
    )(x, W1, degp)


def _tc_mid(p, hp, dinv16, b, W, Fi, Fo):
    def body(p_ref, h_ref, dv_ref, b_ref, w_ref, o_ref):
        dinv = dv_ref[:, :1]
        t = (p_ref[0] + p_ref[1] + h_ref[...]) * dinv + b_ref[...]
        r = jnp.maximum(t, 0.0)
        o_ref[...] = (
            jnp.dot(r, w_ref[...], preferred_element_type=jnp.float32) * dinv
        )

    return pl.pallas_call(
        body,
        grid=(N // BL,),
        in_specs=[
            pl.BlockSpec((2, BL, Fi), lambda i: (0, i, 0)),
            pl.BlockSpec((BL, Fi), lambda i: (i, 0)),
            pl.BlockSpec((BL, 16), lambda i: (i, 0)),
            pl.BlockSpec((1, Fi), lambda i: (0, 0)),
            pl.BlockSpec((Fi, Fo), lambda i: (0, 0)),
        ],
        out_specs=pl.BlockSpec((BL, Fo), lambda i: (i, 0)),
        out_shape=jax.ShapeDtypeStruct((N, Fo), jnp.float32),
    )(p, hp, dinv16, b, W)


def _tc_last(p, hp, dinv16, b):
    def body(p_ref, h_ref, dv_ref, b_ref, o_ref):
        dinv = dv_ref[:, :1]
        o_ref[...] = (p_ref[0] + p_ref[1] + h_ref[...]) * dinv + b_ref[...]

    return pl.pallas_call(
        body,
        grid=(N // BL,),
        in_specs=[
            pl.BlockSpec((2, BL, 16), lambda i: (0, i, 0)),
            pl.BlockSpec((BL, 16), lambda i: (i, 0)),
            pl.BlockSpec((BL, 16), lambda i: (i, 0)),
            pl.BlockSpec((1, 16), lambda i: (0, 0)),
        ],
        out_specs=pl.BlockSpec((BL, 16), lambda i: (i, 0)),
        out_shape=jax.ShapeDtypeStruct((N, 16), jnp.float32),
    )(p, hp, dinv16, b)


def kernel(x, edge_index, W1, b1, Wn, bn, W2, b2):
    src = edge_index[0]
    dst = edge_index[1]
    pad = NW * EPT - E
    src_p = jnp.concatenate([src, jnp.zeros((pad,), jnp.int32)])
    dst_p = jnp.concatenate([dst, jnp.full((pad,), N, jnp.int32)])
    srcr = src_p.reshape(NW, NCHUNK, CHUNK)
    dstr = dst_p.reshape(NW, NCHUNK, CHUNK)
    ones_deg = jnp.ones((CHUNK, DF), jnp.float32)
    zeros_deg = jnp.zeros((RPT, DF), jnp.float32)
    zeros16 = jnp.zeros((RPT, 16), jnp.float32)
    zeros32 = jnp.zeros((RPT, 32), jnp.float32)

    degp = _deg_kernel(dstr, ones_deg, zeros_deg)
    h1p, dinv16 = _tc_first(x, W1, degp)
    p1 = _agg32(h1p, srcr, dstr, zeros32)
    h2p = _tc_mid(p1, h1p, dinv16, b1.reshape(1, -1), Wn, 32, 32)
    p2 = _agg32(h2p, srcr, dstr, zeros32)
    h3p = _tc_mid(p2, h2p, dinv16, bn.reshape(1, -1), W2, 32, 16)
    p3 = _agg16(h3p, srcr, dstr, zeros16)
    return _tc_last(p3, h3p, dinv16, b2.reshape(1, -1))


# speedup vs baseline: 46.1426x; 1.0678x over previous
import functools

import jax
import jax.numpy as jnp
from jax import lax
from jax.experimental import pallas as pl
from jax.experimental.pallas import tpu as pltpu
from jax.experimental.pallas import tpu_sc as plsc

N = 10000
E = 320000
NC, NS = 2, 16
NW = NC * NS
CHUNK = 128
EPT = 10240
NCHUNK = EPT // CHUNK
ACC = 10240
RPT = ACC // NS
BL = 2000
DF = 8


def _make_mesh():
    return plsc.VectorSubcoreMesh(
        core_axis_name="c", subcore_axis_name="s", num_cores=NC, num_subcores=NS
    )


def _make_agg(F):
    @functools.partial(
        pl.kernel,
        out_type=jax.ShapeDtypeStruct((NC, ACC, F), jnp.float32),
        mesh=_make_mesh(),
        scratch_types=[
            pltpu.VMEM((NCHUNK, CHUNK), jnp.int32),
            pltpu.VMEM((NCHUNK, CHUNK), jnp.int32),
            pltpu.VMEM((4, CHUNK, F), jnp.float32),
            pltpu.VMEM((RPT, F), jnp.float32),
            pltpu.VMEM_SHARED((ACC, F), jnp.float32),
            pltpu.VMEM_SHARED((ACC, F), jnp.float32),
            [pltpu.SemaphoreType.DMA] * 4,
        ],
        compiler_params=pltpu.CompilerParams(use_tc_tiling_on_sc=False, skip_device_barrier=True),
    )
    def agg(hp, eir, zeros, out, src_v, dst_v, gbuf, zbuf, acc,
            hp_sh, gsems):
        c = lax.axis_index("c")
        s = lax.axis_index("s")
        wid = s * NC + c
        pltpu.sync_copy(zeros, zbuf)
        pltpu.sync_copy(zbuf, acc.at[pl.ds(s * RPT, RPT)])
        pltpu.sync_copy(hp.at[pl.ds(s * RPT, RPT)],
                        hp_sh.at[pl.ds(s * RPT, RPT)])
        pltpu.sync_copy(eir.at[0, wid], src_v)
        pltpu.sync_copy(eir.at[1, wid], dst_v)
        plsc.subcore_barrier()

        def start(j, slot):
            pltpu.async_copy(hp_sh.at[src_v.at[j]], gbuf.at[slot],
                             gsems[slot])

        def finish(j, slot):
            pltpu.make_async_copy(hp_sh.at[src_v.at[j]], gbuf.at[slot],
                                  gsems[slot]).wait()
            pltpu.sync_copy(gbuf.at[slot], acc.at[dst_v.at[j]], add=True)

        for t in range(4):
            start(t, t)

        def body(k, carry):
            j = 4 * k
            for t in range(4):
                finish(j + t, t)

                @pl.when(j + t + 4 < NCHUNK)
                def _():
                    start(j + t + 4, t)

            return carry

        lax.fori_loop(0, NCHUNK // 4, body, 0)
        plsc.subcore_barrier()
        pltpu.sync_copy(acc.at[pl.ds(s * RPT, RPT)],
                        out.at[c, pl.ds(s * RPT, RPT)])

    return agg


def _make_deg():
    @functools.partial(
        pl.kernel,
        out_type=jax.ShapeDtypeStruct((NC, ACC, DF), jnp.float32),
        mesh=_make_mesh(),
        scratch_types=[
            pltpu.VMEM((NCHUNK, CHUNK), jnp.int32),
            pltpu.VMEM((CHUNK, DF), jnp.float32),
            pltpu.VMEM((RPT, DF), jnp.float32),
            pltpu.VMEM_SHARED((ACC, DF), jnp.float32),
        ],
        compiler_params=pltpu.CompilerParams(use_tc_tiling_on_sc=False, skip_device_barrier=True),
    )
    def deg(eir, ones, zeros, out, dst_v, ones_v, zbuf, acc):
        c = lax.axis_index("c")
        s = lax.axis_index("s")
        wid = s * NC + c
        pltpu.sync_copy(zeros, zbuf)
        pltpu.sync_copy(zbuf, acc.at[pl.ds(s * RPT, RPT)])
        pltpu.sync_copy(eir.at[1, wid], dst_v)
        pltpu.sync_copy(ones, ones_v)
        plsc.subcore_barrier()

        def body(j, carry):
            pltpu.sync_copy(ones_v, acc.at[dst_v.at[j]], add=True)
            return carry

        lax.fori_loop(0, NCHUNK, body, 0)
        plsc.subcore_barrier()
        pltpu.sync_copy(acc.at[pl.ds(s * RPT, RPT)],
                        out.at[c, pl.ds(s * RPT, RPT)])

    return deg


_deg_kernel = _make_deg()
_agg32 = _make_agg(32)
_agg16 = _make_agg(16)


def _tc_first(x, W1, degp):
    def body(x_ref, w_ref, dp_ref, h_ref, dv_ref):
        deg = jnp.sum(dp_ref[..., 0], axis=0) + 1.0
        dinv = lax.rsqrt(deg)
        h = jnp.dot(x_ref[...], w_ref[...], preferred_element_type=jnp.float32)
        h_ref[...] = h * dinv[:, None]
        dv_ref[...] = jnp.broadcast_to(dinv[:, None], (BL, 16))

    return pl.pallas_call(
        body,
        grid=(N // BL,),
        in_specs=[
            pl.BlockSpec((BL, 128), lambda i: (i, 0)),
            pl.BlockSpec((128, 32), lambda i: (0, 0)),
            pl.BlockSpec((2, BL, DF), lambda i: (0, i, 0)),
        ],
        out_specs=[
            pl.BlockSpec((BL, 32), lambda i: (i, 0)),
            pl.BlockSpec((BL, 16), lambda i: (i, 0)),
        ],
        out_shape=[
            jax.ShapeDtypeStruct((ACC, 32), jnp.float32),
            jax.ShapeDtypeStruct((N, 16), jnp.float32),
        ],
    )(x, W1, degp)


def _tc_mid(p, hp, dinv16, b, W, Fi, Fo):
    def body(p_ref, h_ref, dv_ref, b_ref, w_ref, o_ref):
        dinv = dv_ref[:, :1]
        t = (p_ref[0] + p_ref[1] + h_ref[...]) * dinv + b_ref[...]
        r = jnp.maximum(t, 0.0)
        o_ref[...] = (
            jnp.dot(r, w_ref[...], preferred_element_type=jnp.float32) * dinv
        )

    return pl.pallas_call(
        body,
        grid=(N // BL,),
        in_specs=[
            pl.BlockSpec((2, BL, Fi), lambda i: (0, i, 0)),
            pl.BlockSpec((BL, Fi), lambda i: (i, 0)),
            pl.BlockSpec((BL, 16), lambda i: (i, 0)),
            pl.BlockSpec((1, Fi), lambda i: (0, 0)),
            pl.BlockSpec((Fi, Fo), lambda i: (0, 0)),
        ],
        out_specs=pl.BlockSpec((BL, Fo), lambda i: (i, 0)),
        out_shape=jax.ShapeDtypeStruct((ACC, Fo), jnp.float32),
    )(p, hp, dinv16, b, W)


def _tc_last(p, hp, dinv16, b):
    def body(p_ref, h_ref, dv_ref, b_ref, o_ref):
        dinv = dv_ref[:, :1]
        o_ref[...] = (p_ref[0] + p_ref[1] + h_ref[...]) * dinv + b_ref[...]

    return pl.pallas_call(
        body,
        grid=(N // BL,),
        in_specs=[
            pl.BlockSpec((2, BL, 16), lambda i: (0, i, 0)),
            pl.BlockSpec((BL, 16), lambda i: (i, 0)),
            pl.BlockSpec((BL, 16), lambda i: (i, 0)),
            pl.BlockSpec((1, 16), lambda i: (0, 0)),
        ],
        out_specs=pl.BlockSpec((BL, 16), lambda i: (i, 0)),
        out_shape=jax.ShapeDtypeStruct((N, 16), jnp.float32),
    )(p, hp, dinv16, b)


def kernel(x, edge_index, W1, b1, Wn, bn, W2, b2):
    pad = NW * EPT - E
    pad2 = jnp.stack([jnp.zeros((pad,), jnp.int32),
                      jnp.full((pad,), N, jnp.int32)])
    eir = jnp.concatenate([edge_index, pad2], axis=1).reshape(
        2, NW, NCHUNK, CHUNK)
    ones_deg = jnp.ones((CHUNK, DF), jnp.float32)
    zeros_deg = jnp.zeros((RPT, DF), jnp.float32)
    zeros16 = jnp.zeros((RPT, 16), jnp.float32)
    zeros32 = jnp.zeros((RPT, 32), jnp.float32)

    degp = _deg_kernel(eir, ones_deg, zeros_deg)
    h1p, dinv16 = _tc_first(x, W1, degp)
    p1 = _agg32(h1p, eir, zeros32)
    h2p = _tc_mid(p1, h1p, dinv16, b1.reshape(1, -1), Wn, 32, 32)
    p2 = _agg32(h2p, eir, zeros32)
    h3p = _tc_mid(p2, h2p, dinv16, bn.reshape(1, -1), W2, 32, 16)
    p3 = _agg16(h3p, eir, zeros16)
    return _tc_last(p3, h3p, dinv16, b2.reshape(1, -1))
